# baseline (speedup 1.0000x reference)
import functools

import jax
import jax.numpy as jnp
from jax import lax
from jax.experimental import pallas as pl
from jax.experimental.pallas import tpu as pltpu
from jax.experimental.pallas import tpu_sc as plsc

B = 16384
M = 1000000
NW = 16
BP = B // NW
NCH = BP // 128
L = 16

_mesh = plsc.VectorSubcoreMesh(core_axis_name="c", subcore_axis_name="s",
                               num_cores=1)


@functools.partial(
    pl.kernel,
    out_type=jax.ShapeDtypeStruct((B,), jnp.float32),
    mesh=_mesh,
    scratch_types=[
        pltpu.VMEM((BP,), jnp.float32),
        pltpu.VMEM((NCH, 128), jnp.int32),
        pltpu.VMEM((NCH, 128), jnp.float32),
        pltpu.VMEM((BP,), jnp.float32),
        pltpu.VMEM((L,), jnp.float32),
        pltpu.SemaphoreType.DMA,
        pltpu.SemaphoreType.DMA,
    ],
)
def _sc_body(loss_hbm, idx_hbm, eta_hbm, w_ref, out_hbm,
             loss_v, idx_v, w_v, o_v, eta_v, in_sem, out_sem):
    wid = lax.axis_index("s")
    base = wid * BP

    c_loss = pltpu.async_copy(loss_hbm.at[pl.ds(base, BP)], loss_v, in_sem)
    c_idx = pltpu.async_copy(idx_hbm.at[pl.ds(wid * NCH, NCH), :], idx_v,
                             in_sem)
    c_eta = pltpu.async_copy(eta_hbm, eta_v, in_sem)
    c_loss.wait()
    c_idx.wait()
    c_eta.wait()

    eta = eta_v[...]
    for i in range(BP // L):
        lv = loss_v[pl.ds(i * L, L)]
        t = 1.0 - lv / eta
        s = 1.0 / (1.0 + jnp.exp(-t))
        w = jnp.where(lv > eta, 0.0, s)
        o = 1.0 / (1.0 + jnp.exp(-w))
        j, r = (i * L) // 128, (i * L) % 128
        w_v[j, pl.ds(r, L)] = w
        o_v[pl.ds(i * L, L)] = o

    copies = [pltpu.async_copy(o_v, out_hbm.at[pl.ds(base, BP)], out_sem)]
    copies += [pltpu.async_copy(w_v.at[j], w_ref.at[idx_v.at[j]], out_sem)
               for j in range(NCH)]
    for c in copies:
        c.wait()


def kernel(loss, idx, weights, eta_value):
    eta16 = jnp.broadcast_to(eta_value, (L,))
    idx2d = idx.reshape(NW * NCH, 128)
    w_ref = jax.new_ref(weights)
    out = _sc_body(loss, idx2d, eta16, w_ref)
    return out, jax.freeze(w_ref)


# speedup vs baseline: 1.3391x; 1.3391x over previous
import functools

import jax
import jax.numpy as jnp
from jax import lax
from jax.experimental import pallas as pl
from jax.experimental.pallas import tpu as pltpu
from jax.experimental.pallas import tpu_sc as plsc

B = 16384
M = 1000000
NSC = 2
NT = 16
Mh = M // NSC
CH = 31248
EXTRA = Mh - NT * CH
DWIN = 128
REG = Mh + NT * DWIN + 16
BPT = B // NT
NR = BPT // 128
L = 16

_mesh = plsc.VectorSubcoreMesh(core_axis_name="c", subcore_axis_name="s")


@functools.partial(
    pl.kernel,
    out_type=[jax.ShapeDtypeStruct((B,), jnp.float32),
              jax.ShapeDtypeStruct((M,), jnp.float32)],
    mesh=_mesh,
    scratch_types=[
        pltpu.VMEM((BPT,), jnp.float32),
        pltpu.VMEM((NR, 128), jnp.int32),
        pltpu.VMEM((NR, 128), jnp.float32),
        pltpu.VMEM((BPT,), jnp.float32),
        pltpu.VMEM((L,), jnp.float32),
        pltpu.VMEM((CH,), jnp.float32),
        pltpu.VMEM((EXTRA,), jnp.float32),
        pltpu.VMEM_SHARED((REG,), jnp.float32),
        pltpu.SemaphoreType.DMA,
        pltpu.SemaphoreType.DMA,
    ],
)
def _sc_body(loss_hbm, idx_hbm, eta_hbm, w_hbm, out_hbm, neww_hbm,
             loss_v, li_v, w_v, o_v, eta_v, buf, tbuf, region,
             in_sem, reg_sem):
    c = lax.axis_index("c")
    s = lax.axis_index("s")
    R = c * Mh
    base = s * BPT
    rb = s * CH

    c_loss = pltpu.async_copy(loss_hbm.at[pl.ds(base, BPT)], loss_v, in_sem)
    c_idx = pltpu.async_copy(idx_hbm.at[pl.ds(s * NR, NR), :], li_v, in_sem)
    c_eta = pltpu.async_copy(eta_hbm, eta_v, in_sem)
    r_in = pltpu.async_copy(w_hbm.at[pl.ds(R + rb, CH)], buf, reg_sem)
    c_loss.wait()
    c_idx.wait()
    c_eta.wait()

    eta = eta_v[...]
    dummy = Mh + s * DWIN
    for i in range(BPT // L):
        j, r = (i * L) // 128, (i * L) % 128
        lv = loss_v[pl.ds(i * L, L)]
        iv = li_v[j, pl.ds(r, L)]
        t = 1.0 - lv / eta
        sg = 1.0 / (1.0 + jnp.exp(-t))
        w = jnp.where(lv > eta, 0.0, sg)
        o = 1.0 / (1.0 + jnp.exp(-w))
        in_half = (iv >= R) & (iv < R + Mh)
        li = jnp.where(in_half, iv - R, dummy + i % DWIN)
        w_v[j, pl.ds(r, L)] = w
        li_v[j, pl.ds(r, L)] = li
        o_v[pl.ds(i * L, L)] = o

    @pl.when(c == 0)
    def _():
        pltpu.sync_copy(o_v, out_hbm.at[pl.ds(base, BPT)])

    r_in.wait()
    pltpu.sync_copy(buf, region.at[pl.ds(rb, CH)])

    @pl.when(s == 0)
    def _():
        pltpu.sync_copy(w_hbm.at[pl.ds(R + NT * CH, EXTRA)], tbuf)
        pltpu.sync_copy(tbuf, region.at[pl.ds(NT * CH, EXTRA)])

    plsc.subcore_barrier()

    scat = [pltpu.async_copy(w_v.at[j], region.at[li_v.at[j]], reg_sem)
            for j in range(NR)]
    for cp in scat:
        cp.wait()
    plsc.subcore_barrier()

    pltpu.sync_copy(region.at[pl.ds(rb, CH)], buf)
    wb = pltpu.async_copy(buf, neww_hbm.at[pl.ds(R + rb, CH)], reg_sem)

    @pl.when(s == 0)
    def _():
        pltpu.sync_copy(region.at[pl.ds(NT * CH, EXTRA)], tbuf)
        pltpu.sync_copy(tbuf, neww_hbm.at[pl.ds(R + NT * CH, EXTRA)])

    wb.wait()


def kernel(loss, idx, weights, eta_value):
    eta16 = jnp.broadcast_to(eta_value, (L,))
    idx2d = idx.reshape(B // 128, 128)
    out, new_weights = _sc_body(loss, idx2d, eta16, weights)
    return (out, new_weights)


# speedup vs baseline: 1.3590x; 1.0148x over previous
import functools

import jax
import jax.numpy as jnp
from jax import lax
from jax.experimental import pallas as pl
from jax.experimental.pallas import tpu as pltpu
from jax.experimental.pallas import tpu_sc as plsc

B = 16384
M = 1000000
NSC = 2
NT = 16
Mh = M // NSC
CH = 31232
NSUB = 4
SUB = CH // NSUB
EXTRA = Mh - NT * CH
DWIN = 128
REG = Mh + NT * DWIN + 16
BPT = B // NT
NR = BPT // 128
L = 16

_mesh = plsc.VectorSubcoreMesh(core_axis_name="c", subcore_axis_name="s")


@functools.partial(
    pl.kernel,
    out_type=[jax.ShapeDtypeStruct((B,), jnp.float32),
              jax.ShapeDtypeStruct((M,), jnp.float32)],
    mesh=_mesh,
    scratch_types=[
        pltpu.VMEM((BPT,), jnp.float32),
        pltpu.VMEM((NR, 128), jnp.int32),
        pltpu.VMEM((NR, 128), jnp.float32),
        pltpu.VMEM((BPT,), jnp.float32),
        pltpu.VMEM((L,), jnp.float32),
        pltpu.VMEM((CH,), jnp.float32),
        pltpu.VMEM((EXTRA,), jnp.float32),
        pltpu.VMEM_SHARED((REG,), jnp.float32),
        pltpu.SemaphoreType.DMA,
        pltpu.SemaphoreType.DMA,
    ],
)
def _sc_body(loss_hbm, idx_hbm, eta_hbm, w_hbm, out_hbm, neww_hbm,
             loss_v, li_v, w_v, o_v, eta_v, buf, tbuf, region,
             in_sem, reg_sem):
    c = lax.axis_index("c")
    s = lax.axis_index("s")
    R = c * Mh
    base = s * BPT
    rb = s * CH

    c_loss = pltpu.async_copy(loss_hbm.at[pl.ds(base, BPT)], loss_v, in_sem)
    c_idx = pltpu.async_copy(idx_hbm.at[pl.ds(s * NR, NR), :], li_v, in_sem)
    c_eta = pltpu.async_copy(eta_hbm, eta_v, in_sem)
    r_in = [pltpu.async_copy(w_hbm.at[pl.ds(R + rb + k * SUB, SUB)],
                             buf.at[pl.ds(k * SUB, SUB)], reg_sem)
            for k in range(NSUB)]
    c_loss.wait()
    c_idx.wait()
    c_eta.wait()

    eta = eta_v[...]
    dummy = Mh + s * DWIN
    for i in range(BPT // L):
        j, r = (i * L) // 128, (i * L) % 128
        lv = loss_v[pl.ds(i * L, L)]
        iv = li_v[j, pl.ds(r, L)]
        t = 1.0 - lv / eta
        sg = 1.0 / (1.0 + jnp.exp(-t))
        w = jnp.where(lv > eta, 0.0, sg)
        o = 1.0 / (1.0 + jnp.exp(-w))
        in_half = (iv >= R) & (iv < R + Mh)
        li = jnp.where(in_half, iv - R, dummy + i % DWIN)
        w_v[j, pl.ds(r, L)] = w
        li_v[j, pl.ds(r, L)] = li
        o_v[pl.ds(i * L, L)] = o

    @pl.when(c == 0)
    def _():
        pltpu.sync_copy(o_v, out_hbm.at[pl.ds(base, BPT)])

    mids = []
    for k in range(NSUB):
        r_in[k].wait()
        mids.append(pltpu.async_copy(buf.at[pl.ds(k * SUB, SUB)],
                                     region.at[pl.ds(rb + k * SUB, SUB)],
                                     reg_sem))
    for cp in mids:
        cp.wait()

    @pl.when(s == 0)
    def _():
        pltpu.sync_copy(w_hbm.at[pl.ds(R + NT * CH, EXTRA)], tbuf)
        pltpu.sync_copy(tbuf, region.at[pl.ds(NT * CH, EXTRA)])

    plsc.subcore_barrier()

    scat = [pltpu.async_copy(w_v.at[j], region.at[li_v.at[j]], reg_sem)
            for j in range(NR)]
    for cp in scat:
        cp.wait()
    plsc.subcore_barrier()

    backs = [pltpu.async_copy(region.at[pl.ds(rb + k * SUB, SUB)],
                              buf.at[pl.ds(k * SUB, SUB)], in_sem)
             for k in range(NSUB)]
    wbs = []
    for k in range(NSUB):
        backs[k].wait()
        wbs.append(pltpu.async_copy(buf.at[pl.ds(k * SUB, SUB)],
                                    neww_hbm.at[pl.ds(R + rb + k * SUB, SUB)],
                                    reg_sem))

    @pl.when(s == 0)
    def _():
        pltpu.sync_copy(region.at[pl.ds(NT * CH, EXTRA)], tbuf)
        pltpu.sync_copy(tbuf, neww_hbm.at[pl.ds(R + NT * CH, EXTRA)])

    for cp in wbs:
        cp.wait()


def kernel(loss, idx, weights, eta_value):
    eta16 = jnp.broadcast_to(eta_value, (L,))
    idx2d = idx.reshape(B // 128, 128)
    out, new_weights = _sc_body(loss, idx2d, eta16, weights)
    return (out, new_weights)


# speedup vs baseline: 1.3590x; 1.0000x over previous
import functools

import jax
import jax.numpy as jnp
from jax import lax
from jax.experimental import pallas as pl
from jax.experimental.pallas import tpu as pltpu
from jax.experimental.pallas import tpu_sc as plsc

B = 16384
M = 1000000
NSC = 2
NT = 16
Mh = M // NSC
CH = 31232
NSUB = 4
SUB = CH // NSUB
EXTRA = Mh - NT * CH
DWIN = 128
REG = Mh + NT * DWIN + 16
BPT = B // NT
NR = BPT // 128
L = 16

_mesh = plsc.VectorSubcoreMesh(core_axis_name="c", subcore_axis_name="s")


@functools.partial(
    pl.kernel,
    out_type=[jax.ShapeDtypeStruct((B,), jnp.float32),
              jax.ShapeDtypeStruct((M,), jnp.float32)],
    mesh=_mesh,
    scratch_types=[
        pltpu.VMEM((BPT,), jnp.float32),
        pltpu.VMEM((NR, 128), jnp.int32),
        pltpu.VMEM((NR, 128), jnp.float32),
        pltpu.VMEM((BPT,), jnp.float32),
        pltpu.VMEM((L,), jnp.float32),
        pltpu.VMEM((CH,), jnp.float32),
        pltpu.VMEM((EXTRA,), jnp.float32),
        pltpu.VMEM_SHARED((REG,), jnp.float32),
        pltpu.SemaphoreType.DMA,
        pltpu.SemaphoreType.DMA,
    ],
)
def _sc_body(loss_hbm, idx_hbm, eta_hbm, w_hbm, out_hbm, neww_hbm,
             loss_v, li_v, w_v, o_v, eta_v, buf, tbuf, region,
             in_sem, reg_sem):
    c = lax.axis_index("c")
    s = lax.axis_index("s")
    R = c * Mh
    base = s * BPT
    rb = s * CH

    c_loss = pltpu.async_copy(loss_hbm.at[pl.ds(base, BPT)], loss_v, in_sem)
    c_idx = pltpu.async_copy(idx_hbm.at[pl.ds(s * NR, NR), :], li_v, in_sem)
    c_eta = pltpu.async_copy(eta_hbm, eta_v, in_sem)
    r_in = [pltpu.async_copy(w_hbm.at[pl.ds(R + rb + k * SUB, SUB)],
                             buf.at[pl.ds(k * SUB, SUB)], reg_sem)
            for k in range(NSUB)]
    with jax.named_scope("ph_inwait"):
        c_loss.wait()
        c_idx.wait()
        c_eta.wait()

    eta = eta_v[...]
    dummy = Mh + s * DWIN
    for i in range(BPT // L):
        j, r = (i * L) // 128, (i * L) % 128
        lv = loss_v[pl.ds(i * L, L)]
        iv = li_v[j, pl.ds(r, L)]
        t = 1.0 - lv / eta
        sg = 1.0 / (1.0 + jnp.exp(-t))
        w = jnp.where(lv > eta, 0.0, sg)
        o = 1.0 / (1.0 + jnp.exp(-w))
        in_half = (iv >= R) & (iv < R + Mh)
        li = jnp.where(in_half, iv - R, dummy + i % DWIN)
        w_v[j, pl.ds(r, L)] = w
        li_v[j, pl.ds(r, L)] = li
        o_v[pl.ds(i * L, L)] = o

    with jax.named_scope("ph_outw"):
        @pl.when(c == 0)
        def _():
            pltpu.sync_copy(o_v, out_hbm.at[pl.ds(base, BPT)])

    with jax.named_scope("ph_stage"):
        mids = []
        for k in range(NSUB):
            r_in[k].wait()
            mids.append(pltpu.async_copy(buf.at[pl.ds(k * SUB, SUB)],
                                         region.at[pl.ds(rb + k * SUB, SUB)],
                                         reg_sem))
        for cp in mids:
            cp.wait()

        @pl.when(s == 0)
        def _():
            pltpu.sync_copy(w_hbm.at[pl.ds(R + NT * CH, EXTRA)], tbuf)
            pltpu.sync_copy(tbuf, region.at[pl.ds(NT * CH, EXTRA)])

    with jax.named_scope("ph_bar1"):
        plsc.subcore_barrier()

    with jax.named_scope("ph_scat"):
        scat = [pltpu.async_copy(w_v.at[j], region.at[li_v.at[j]], reg_sem)
                for j in range(NR)]
        for cp in scat:
            cp.wait()
    with jax.named_scope("ph_bar2"):
        plsc.subcore_barrier()

    with jax.named_scope("ph_wb"):
        backs = [pltpu.async_copy(region.at[pl.ds(rb + k * SUB, SUB)],
                                  buf.at[pl.ds(k * SUB, SUB)], in_sem)
                 for k in range(NSUB)]
        wbs = []
        for k in range(NSUB):
            backs[k].wait()
            wbs.append(pltpu.async_copy(buf.at[pl.ds(k * SUB, SUB)],
                                        neww_hbm.at[pl.ds(R + rb + k * SUB, SUB)],
                                        reg_sem))

        @pl.when(s == 0)
        def _():
            pltpu.sync_copy(region.at[pl.ds(NT * CH, EXTRA)], tbuf)
            pltpu.sync_copy(tbuf, neww_hbm.at[pl.ds(R + NT * CH, EXTRA)])

        for cp in wbs:
            cp.wait()


def kernel(loss, idx, weights, eta_value):
    eta16 = jnp.broadcast_to(eta_value, (L,))
    idx2d = idx.reshape(B // 128, 128)
    out, new_weights = _sc_body(loss, idx2d, eta16, weights)
    return (out, new_weights)
